# baseline (speedup 1.0000x reference)
import jax
import jax.numpy as jnp
from jax.experimental import pallas as pl
from jax.experimental.pallas import tpu as pltpu

_N = 128
_C = 64
_S = 25
_K = 26
_EPS_DIV = 2.220446049250313e-16
_BN_S = 1.0 / (1.0 + 1e-5) ** 0.5


def _body(xs_ref, w1m_ref, scale1_ref, cb1_ref, w2big_ref, fc3w_ref,
          fc3b_ref, fc4wp_ref, feat_ref, gcnw_ref, s2_ref, cbg_ref,
          scal_ref, out_ref):
    f32 = jnp.float32
    sc2 = scal_ref[0]
    cb2 = scal_ref[1]
    cb4 = scal_ref[2]
    a0 = scal_ref[3]
    a1 = scal_ref[4]
    a2 = scal_ref[5]

    xs = xs_ref[...]
    srow = jax.lax.broadcasted_iota(jnp.int32, (_S * _N, _C), 0) // _N
    si = srow // 5
    sj = srow - 5 * si
    y1 = None
    for di in range(3):
        for dj in range(3):
            off = di * 3 + dj
            dshift = (di - 1) * 5 + (dj - 1)
            xsh = jnp.roll(xs, -dshift * _N, axis=0) if dshift else xs
            ii = si + (di - 1)
            jj = sj + (dj - 1)
            valid = (ii >= 0) & (ii < 5) & (jj >= 0) & (jj < 5)
            xm = jnp.where(valid, xsh, 0.0)
            w = w1m_ref[off * _C:(off + 1) * _C, :]
            t = jax.lax.dot_general(xm, w, (((1,), (0,)), ((), ())),
                                    preferred_element_type=f32)
            y1 = t if y1 is None else y1 + t
    y1 = jnp.maximum(y1 * scale1_ref[...] + cb1_ref[...], 0.0)

    win = {0: (0,), 1: (1, 2), 2: (3, 4)}
    p = []
    for i2 in range(3):
        for j2 in range(3):
            m = None
            for i in win[i2]:
                for j in win[j2]:
                    b = y1[(5 * i + j) * _N:(5 * i + j + 1) * _N, :]
                    m = b if m is None else jnp.maximum(m, b)
            p.append(m)

    pcat = jnp.concatenate(p, axis=1)
    z = jax.lax.dot_general(pcat, w2big_ref[...], (((1,), (0,)), ((), ())),
                            preferred_element_type=f32)
    z = jnp.maximum(z * sc2 + cb2, 0.0)

    v0 = z[:, 0:1]
    v1 = jnp.maximum(z[:, 1:2], z[:, 2:3])
    v2 = jnp.maximum(z[:, 3:4], z[:, 6:7])
    v3 = jnp.maximum(jnp.maximum(z[:, 4:5], z[:, 5:6]),
                     jnp.maximum(z[:, 7:8], z[:, 8:9]))
    V = jnp.concatenate([v0, v1, v2, v3], axis=1)
    h3 = jnp.maximum(
        jax.lax.dot_general(V, fc3w_ref[...], (((1,), (1,)), ((), ())),
                            preferred_element_type=f32) + fc3b_ref[...], 0.0)
    s16 = jax.lax.dot_general(h3, fc4wp_ref[...], (((1,), (1,)), ((), ())),
                              preferred_element_type=f32)
    ci16 = jax.lax.broadcasted_iota(jnp.int32, (_N, 16), 1)
    sigma = jnp.sum(jnp.where(ci16 == 0, s16, 0.0), axis=1, keepdims=True)
    rec = 1.0 / (sigma + cb4)

    feats = feat_ref[...]
    f = feats * rec
    G = jax.lax.dot_general(f, f, (((1,), (1,)), ((), ())),
                            preferred_element_type=f32)
    ri = jax.lax.broadcasted_iota(jnp.int32, (_N, _N), 0)
    ci = jax.lax.broadcasted_iota(jnp.int32, (_N, _N), 1)
    eye = (ri == ci).astype(f32)
    gd = G * eye
    nv_col = jnp.sum(gd, axis=1, keepdims=True)
    nv_row = jnp.sum(gd, axis=0, keepdims=True)
    t2 = jnp.maximum(nv_col + nv_row - 2.0 * G, 0.0)
    ae = jnp.exp(-t2)

    def topk_step(_, carry):
        w_, msk = carry
        m = jnp.max(w_, axis=1, keepdims=True)
        cand = w_ == m
        idx = jnp.min(jnp.where(cand, ci, _N), axis=1, keepdims=True)
        sel = ci == idx
        return jnp.where(sel, -1.0, w_), msk + sel.astype(jnp.float32)

    _, maskf = jax.lax.fori_loop(
        0, _K, topk_step, (ae, jnp.zeros((_N, _N), jnp.float32)))

    ae_m = jnp.where((maskf > 0.0) & (ri != ci), ae, 0.0)
    A = eye + ae_m
    d = jnp.sum(A, axis=1, keepdims=True) + 1.0
    rc = 1.0 / jnp.sqrt(d)
    An = jax.lax.dot_general(A * rc, eye * rc, (((1,), (0,)), ((), ())),
                             preferred_element_type=f32)
    An2 = jax.lax.dot_general(An, An, (((1,), (0,)), ((), ())),
                              preferred_element_type=f32)
    M = a0 * eye + a1 * An + a2 * An2

    support = jax.lax.dot_general(feats, gcnw_ref[...], (((1,), (0,)), ((), ())),
                                  preferred_element_type=f32)
    outv = jax.lax.dot_general(M, support, (((1,), (0,)), ((), ())),
                               preferred_element_type=f32)
    out_ref[...] = jnp.maximum(outv * s2_ref[...] + cbg_ref[...], 0.0)


@jax.jit
def kernel(features, conv1_w, conv1_b, bn_c1_g, bn_c1_b, conv2_w, conv2_b,
           bn_c2_g, bn_c2_b, fc3_w, fc3_b, fc4_w, fc4_b, gcn_w, gcn_b,
           bn2_g, bn2_b, aifa1, aifa2, aifa3):
    xs = features.reshape(_N, _C, _S).transpose(2, 0, 1).reshape(_S * _N, _C)
    w1m = conv1_w.transpose(2, 3, 1, 0).reshape(9 * _C, _C)
    s1 = bn_c1_g * _BN_S
    scale1 = s1.reshape(1, _C)
    cb1 = (conv1_b * s1 + bn_c1_b).reshape(1, _C)

    w2t = conv2_w[0].reshape(_C, 9)
    w2big = jnp.zeros((9 * _C, 16), jnp.float32)
    for i2 in range(3):
        for j2 in range(3):
            t = i2 * 3 + j2
            for di in range(3):
                for dj in range(3):
                    ti, tj = i2 + di - 1, j2 + dj - 1
                    if 0 <= ti < 3 and 0 <= tj < 3:
                        tp = ti * 3 + tj
                        w2big = w2big.at[tp * _C:(tp + 1) * _C, t].set(
                            w2t[:, di * 3 + dj])

    s2c = bn_c2_g[0] * _BN_S
    fc4wp = jnp.zeros((16, 8), jnp.float32).at[0, :].set(fc4_w[0])
    aifa = jax.nn.softmax(jnp.concatenate([aifa1, aifa2, aifa3]))
    scal = jnp.stack([
        s2c,
        conv2_b[0] * s2c + bn_c2_b[0],
        fc4_b[0] + _EPS_DIV,
        aifa[0], aifa[1], aifa[2],
        jnp.float32(0.0), jnp.float32(0.0),
    ])

    sg = bn2_g * _BN_S
    s2_row = sg.reshape(1, -1)
    cbg = (gcn_b * sg + bn2_b).reshape(1, -1)

    nvmem = 13
    return pl.pallas_call(
        _body,
        out_shape=jax.ShapeDtypeStruct((_N, gcn_w.shape[1]), jnp.float32),
        in_specs=[pl.BlockSpec(memory_space=pltpu.VMEM)] * (nvmem - 1) +
                 [pl.BlockSpec(memory_space=pltpu.SMEM)],
    )(xs, w1m, scale1, cb1, w2big, fc3_w, fc3_b.reshape(1, -1), fc4wp,
      features, gcn_w, s2_row, cbg, scal)


# speedup vs baseline: 1.9704x; 1.9704x over previous
import jax
import jax.numpy as jnp
import numpy as np
from jax.experimental import pallas as pl
from jax.experimental.pallas import tpu as pltpu

_N = 128
_C = 64
_S = 25
_K = 26
_EPS_DIV = 2.220446049250313e-16
_BN_S = 1.0 / (1.0 + 1e-5) ** 0.5


def _w2sel():
    s = np.zeros((9, 9 * 16), np.float32)
    for i2 in range(3):
        for j2 in range(3):
            t = i2 * 3 + j2
            for di in range(3):
                for dj in range(3):
                    ti, tj = i2 + di - 1, j2 + dj - 1
                    if 0 <= ti < 3 and 0 <= tj < 3:
                        s[di * 3 + dj, (ti * 3 + tj) * 16 + t] = 1.0
    return s


_W2SEL = _w2sel()
_E16 = np.zeros((16, 1), np.float32)
_E16[0, 0] = 1.0


def _body(xs_ref, w1m_ref, scale1_ref, cb1_ref, w2big_ref, fc3w_ref,
          fc3b_ref, fc4wp_ref, feat_ref, gcnw_ref, s2_ref, cbg_ref,
          scal_ref, out_ref):
    f32 = jnp.float32
    sc2 = scal_ref[0]
    cb2 = scal_ref[1]
    cb4 = scal_ref[2]
    a0 = scal_ref[3]
    a1 = scal_ref[4]
    a2 = scal_ref[5]

    xs = xs_ref[...]
    srow = jax.lax.broadcasted_iota(jnp.int32, (_S * _N, _C), 0) // _N
    si = srow // 5
    sj = srow - 5 * si
    y1 = None
    for di in range(3):
        for dj in range(3):
            off = di * 3 + dj
            dshift = (di - 1) * 5 + (dj - 1)
            xsh = jnp.roll(xs, -dshift * _N, axis=0) if dshift else xs
            ii = si + (di - 1)
            jj = sj + (dj - 1)
            valid = (ii >= 0) & (ii < 5) & (jj >= 0) & (jj < 5)
            xm = jnp.where(valid, xsh, 0.0)
            w = w1m_ref[off * _C:(off + 1) * _C, :]
            t = jax.lax.dot_general(xm, w, (((1,), (0,)), ((), ())),
                                    preferred_element_type=f32)
            y1 = t if y1 is None else y1 + t
    y1 = jnp.maximum(y1 * scale1_ref[...] + cb1_ref[...], 0.0)

    win = {0: (0,), 1: (1, 2), 2: (3, 4)}
    p = []
    for i2 in range(3):
        for j2 in range(3):
            m = None
            for i in win[i2]:
                for j in win[j2]:
                    b = y1[(5 * i + j) * _N:(5 * i + j + 1) * _N, :]
                    m = b if m is None else jnp.maximum(m, b)
            p.append(m)

    pcat = jnp.concatenate(p, axis=1)
    z = jax.lax.dot_general(pcat, w2big_ref[...], (((1,), (0,)), ((), ())),
                            preferred_element_type=f32)
    z = jnp.maximum(z * sc2 + cb2, 0.0)

    v0 = z[:, 0:1]
    v1 = jnp.maximum(z[:, 1:2], z[:, 2:3])
    v2 = jnp.maximum(z[:, 3:4], z[:, 6:7])
    v3 = jnp.maximum(jnp.maximum(z[:, 4:5], z[:, 5:6]),
                     jnp.maximum(z[:, 7:8], z[:, 8:9]))
    V = jnp.concatenate([v0, v1, v2, v3], axis=1)
    h3 = jnp.maximum(
        jax.lax.dot_general(V, fc3w_ref[...], (((1,), (1,)), ((), ())),
                            preferred_element_type=f32) + fc3b_ref[...], 0.0)
    s16 = jax.lax.dot_general(h3, fc4wp_ref[...], (((1,), (1,)), ((), ())),
                              preferred_element_type=f32)
    ci16 = jax.lax.broadcasted_iota(jnp.int32, (_N, 16), 1)
    sigma = jnp.sum(jnp.where(ci16 == 0, s16, 0.0), axis=1, keepdims=True)
    rec = 1.0 / (sigma + cb4)

    feats = feat_ref[...]
    f = feats * rec
    G = jax.lax.dot_general(f, f, (((1,), (1,)), ((), ())),
                            preferred_element_type=f32)
    ri = jax.lax.broadcasted_iota(jnp.int32, (_N, _N), 0)
    ci = jax.lax.broadcasted_iota(jnp.int32, (_N, _N), 1)
    eye = (ri == ci).astype(f32)
    gd = G * eye
    nv_col = jnp.sum(gd, axis=1, keepdims=True)
    nv_row = jnp.sum(gd, axis=0, keepdims=True)
    t2 = jnp.maximum(nv_col + nv_row - 2.0 * G, 0.0)
    ae = jnp.exp(-t2)

    def topk_step(_, carry):
        w_, msk = carry
        m = jnp.max(w_, axis=1, keepdims=True)
        cand = w_ == m
        idx = jnp.min(jnp.where(cand, ci, _N), axis=1, keepdims=True)
        sel = ci == idx
        return jnp.where(sel, -1.0, w_), msk + sel.astype(jnp.float32)

    _, maskf = jax.lax.fori_loop(
        0, _K, topk_step, (ae, jnp.zeros((_N, _N), jnp.float32)))

    ae_m = jnp.where((maskf > 0.0) & (ri != ci), ae, 0.0)
    A = eye + ae_m
    d = jnp.sum(A, axis=1, keepdims=True) + 1.0
    rc = 1.0 / jnp.sqrt(d)
    An = jax.lax.dot_general(A * rc, eye * rc, (((1,), (0,)), ((), ())),
                             preferred_element_type=f32)
    An2 = jax.lax.dot_general(An, An, (((1,), (0,)), ((), ())),
                              preferred_element_type=f32)
    M = a0 * eye + a1 * An + a2 * An2

    support = jax.lax.dot_general(feats, gcnw_ref[...], (((1,), (0,)), ((), ())),
                                  preferred_element_type=f32)
    outv = jax.lax.dot_general(M, support, (((1,), (0,)), ((), ())),
                               preferred_element_type=f32)
    out_ref[...] = jnp.maximum(outv * s2_ref[...] + cbg_ref[...], 0.0)


@jax.jit
def kernel(features, conv1_w, conv1_b, bn_c1_g, bn_c1_b, conv2_w, conv2_b,
           bn_c2_g, bn_c2_b, fc3_w, fc3_b, fc4_w, fc4_b, gcn_w, gcn_b,
           bn2_g, bn2_b, aifa1, aifa2, aifa3):
    xs = features.reshape(_N, _C, _S).transpose(2, 0, 1).reshape(_S * _N, _C)
    w1m = conv1_w.transpose(2, 3, 1, 0).reshape(9 * _C, _C)
    s1 = bn_c1_g * _BN_S
    scale1 = s1.reshape(1, _C)
    cb1 = (conv1_b * s1 + bn_c1_b).reshape(1, _C)

    w2t = conv2_w[0].reshape(_C, 9)
    w2big = jnp.dot(w2t, _W2SEL).reshape(_C, 9, 16).transpose(1, 0, 2)
    w2big = w2big.reshape(9 * _C, 16)

    s2c = bn_c2_g[0] * _BN_S
    fc4wp = jnp.dot(_E16, fc4_w)
    aifa = jax.nn.softmax(jnp.concatenate([aifa1, aifa2, aifa3]))
    scal = jnp.stack([
        s2c,
        conv2_b[0] * s2c + bn_c2_b[0],
        fc4_b[0] + _EPS_DIV,
        aifa[0], aifa[1], aifa[2],
        jnp.float32(0.0), jnp.float32(0.0),
    ])

    sg = bn2_g * _BN_S
    s2_row = sg.reshape(1, -1)
    cbg = (gcn_b * sg + bn2_b).reshape(1, -1)

    nvmem = 13
    return pl.pallas_call(
        _body,
        out_shape=jax.ShapeDtypeStruct((_N, gcn_w.shape[1]), jnp.float32),
        in_specs=[pl.BlockSpec(memory_space=pltpu.VMEM)] * (nvmem - 1) +
                 [pl.BlockSpec(memory_space=pltpu.SMEM)],
    )(xs, w1m, scale1, cb1, w2big, fc3_w, fc3_b.reshape(1, -1), fc4wp,
      features, gcn_w, s2_row, cbg, scal)


# speedup vs baseline: 2.4516x; 1.2442x over previous
import jax
import jax.numpy as jnp
import numpy as np
from jax.experimental import pallas as pl
from jax.experimental.pallas import tpu as pltpu

_N = 128
_C = 64
_S = 25
_K = 26
_EPS_DIV = 2.220446049250313e-16
_BN_S = 1.0 / (1.0 + 1e-5) ** 0.5


def _w2sel():
    s = np.zeros((9 * 16, 9), np.float32)
    for i2 in range(3):
        for j2 in range(3):
            t = i2 * 3 + j2
            for di in range(3):
                for dj in range(3):
                    ti, tj = i2 + di - 1, j2 + dj - 1
                    if 0 <= ti < 3 and 0 <= tj < 3:
                        s[(ti * 3 + tj) * 16 + t, di * 3 + dj] = 1.0
    return s


_W2SEL = _w2sel()


def _body(xs_ref, w1m_ref, g1_ref, b1c_ref, b1b_ref, w2t_ref, selt_ref,
          fc3w_ref, fc3b_ref, fc4w_ref, feat_ref, gcnw_ref, g2_ref,
          gb_ref, b2_ref, c2g_ref, c2b_ref, c2bb_ref, fc4b_ref, aifa_ref,
          out_ref):
    f32 = jnp.float32

    xs = xs_ref[...]
    srow = jax.lax.broadcasted_iota(jnp.int32, (_S * _N, _C), 0) // _N
    si = srow // 5
    sj = srow - 5 * si
    y1 = None
    for di in range(3):
        for dj in range(3):
            off = di * 3 + dj
            dshift = (di - 1) * 5 + (dj - 1)
            xsh = jnp.roll(xs, -dshift * _N, axis=0) if dshift else xs
            ii = si + (di - 1)
            jj = sj + (dj - 1)
            valid = (ii >= 0) & (ii < 5) & (jj >= 0) & (jj < 5)
            xm = jnp.where(valid, xsh, 0.0)
            w = w1m_ref[off * _C:(off + 1) * _C, :]
            t = jax.lax.dot_general(xm, w, (((1,), (0,)), ((), ())),
                                    preferred_element_type=f32)
            y1 = t if y1 is None else y1 + t
    scale1 = g1_ref[...] * _BN_S
    cb1 = b1c_ref[...] * scale1 + b1b_ref[...]
    y1 = jnp.maximum(y1 * scale1 + cb1, 0.0)

    win = {0: (0,), 1: (1, 2), 2: (3, 4)}
    p = []
    for i2 in range(3):
        for j2 in range(3):
            m = None
            for i in win[i2]:
                for j in win[j2]:
                    b = y1[(5 * i + j) * _N:(5 * i + j + 1) * _N, :]
                    m = b if m is None else jnp.maximum(m, b)
            p.append(m)

    w2t = w2t_ref[...]
    z = None
    for tp in range(9):
        sel = selt_ref[tp * 16:(tp + 1) * 16, :]
        wcol = jax.lax.dot_general(w2t, sel, (((1,), (1,)), ((), ())),
                                   preferred_element_type=f32)
        zt = jax.lax.dot_general(p[tp], wcol, (((1,), (0,)), ((), ())),
                                 preferred_element_type=f32)
        z = zt if z is None else z + zt
    sc2 = c2g_ref[0] * _BN_S
    cb2 = c2b_ref[0] * sc2 + c2bb_ref[0]
    z = jnp.maximum(z * sc2 + cb2, 0.0)

    v0 = z[:, 0:1]
    v1 = jnp.maximum(z[:, 1:2], z[:, 2:3])
    v2 = jnp.maximum(z[:, 3:4], z[:, 6:7])
    v3 = jnp.maximum(jnp.maximum(z[:, 4:5], z[:, 5:6]),
                     jnp.maximum(z[:, 7:8], z[:, 8:9]))
    V = jnp.concatenate([v0, v1, v2, v3], axis=1)
    h3 = jnp.maximum(
        jax.lax.dot_general(V, fc3w_ref[...], (((1,), (1,)), ((), ())),
                            preferred_element_type=f32) + fc3b_ref[...], 0.0)
    sigma = jnp.sum(h3 * fc4w_ref[...], axis=1, keepdims=True)
    rec = 1.0 / (sigma + (fc4b_ref[0] + _EPS_DIV))

    feats = feat_ref[...]
    f = feats * rec
    G = jax.lax.dot_general(f, f, (((1,), (1,)), ((), ())),
                            preferred_element_type=f32)
    ri = jax.lax.broadcasted_iota(jnp.int32, (_N, _N), 0)
    ci = jax.lax.broadcasted_iota(jnp.int32, (_N, _N), 1)
    eye = (ri == ci).astype(f32)
    gd = G * eye
    nv_col = jnp.sum(gd, axis=1, keepdims=True)
    nv_row = jnp.sum(gd, axis=0, keepdims=True)
    t2 = jnp.maximum(nv_col + nv_row - 2.0 * G, 0.0)
    ae = jnp.exp(-t2)

    def drop_max(_, w_):
        m = jnp.max(w_, axis=1, keepdims=True)
        return jnp.where(w_ == m, -1.0, w_)

    wf = jax.lax.fori_loop(0, _K - 1, drop_max, ae)
    thr = jnp.max(wf, axis=1, keepdims=True)
    ae_m = jnp.where((ae >= thr) & (ri != ci), ae, 0.0)

    A = eye + ae_m
    d = jnp.sum(A, axis=1, keepdims=True) + 1.0
    rc = 1.0 / jnp.sqrt(d)
    An = jax.lax.dot_general(A * rc, eye * rc, (((1,), (0,)), ((), ())),
                             preferred_element_type=f32)
    An2 = jax.lax.dot_general(An, An, (((1,), (0,)), ((), ())),
                              preferred_element_type=f32)
    M = aifa_ref[0] * eye + aifa_ref[1] * An + aifa_ref[2] * An2

    support = jax.lax.dot_general(feats, gcnw_ref[...], (((1,), (0,)), ((), ())),
                                  preferred_element_type=f32)
    outv = jax.lax.dot_general(M, support, (((1,), (0,)), ((), ())),
                               preferred_element_type=f32)
    sg = g2_ref[...] * _BN_S
    cbg = gb_ref[...] * sg + b2_ref[...]
    out_ref[...] = jnp.maximum(outv * sg + cbg, 0.0)


@jax.jit
def kernel(features, conv1_w, conv1_b, bn_c1_g, bn_c1_b, conv2_w, conv2_b,
           bn_c2_g, bn_c2_b, fc3_w, fc3_b, fc4_w, fc4_b, gcn_w, gcn_b,
           bn2_g, bn2_b, aifa1, aifa2, aifa3):
    xs = features.reshape(_N, _C, _S).transpose(2, 0, 1).reshape(_S * _N, _C)
    w1m = conv1_w.transpose(2, 3, 1, 0).reshape(9 * _C, _C)
    aifa = jax.nn.softmax(jnp.concatenate([aifa1, aifa2, aifa3]))

    nv = 15
    return pl.pallas_call(
        _body,
        out_shape=jax.ShapeDtypeStruct((_N, gcn_w.shape[1]), jnp.float32),
        in_specs=[pl.BlockSpec(memory_space=pltpu.VMEM)] * nv +
                 [pl.BlockSpec(memory_space=pltpu.SMEM)] * 5,
    )(xs, w1m,
      bn_c1_g.reshape(1, _C), conv1_b.reshape(1, _C), bn_c1_b.reshape(1, _C),
      conv2_w.reshape(_C, 9), jnp.asarray(_W2SEL),
      fc3_w, fc3_b.reshape(1, -1), fc4_w,
      features, gcn_w,
      bn2_g.reshape(1, -1), gcn_b.reshape(1, -1), bn2_b.reshape(1, -1),
      bn_c2_g, conv2_b, bn_c2_b, fc4_b, aifa)
